# initial kernel scaffold (unmeasured)
import jax
import jax.numpy as jnp
from jax import lax
from jax.experimental import pallas as pl
from jax.experimental.pallas import tpu as pltpu

N_DEV = 8
HEADS_PER = 8
DH = 64
B = 2
SKV = 512


def kernel(x, Wq, K_ext, V_ext, Wo):
    _, sq_loc, d_model = x.shape
    d_chunk = Wq.shape[1]

    k_t = jnp.transpose(K_ext, (2, 0, 1, 3)).astype(jnp.bfloat16)
    v_t = jnp.transpose(V_ext, (2, 0, 1, 3)).astype(jnp.bfloat16)
    x_bf = x.astype(jnp.bfloat16)
    wq_bf = Wq.astype(jnp.bfloat16)
    wo_bf = Wo.astype(jnp.bfloat16)

    def nt_dot(a, b):
        return lax.dot_general(
            a, b, (((1,), (1,)), ((), ())), preferred_element_type=jnp.float32
        )

    def body(x_ref, wq_ref, kt_ref, vt_ref, wo_ref, out_ref,
             wq_comm, wo_comm, wq_send, wq_recv, wo_send, wo_recv):
        my = lax.axis_index("i")
        left = lax.rem(my + N_DEV - 1, N_DEV)
        right = lax.rem(my + 1, N_DEV)

        barrier_sem = pltpu.get_barrier_semaphore()
        for nbr in (left, right):
            pl.semaphore_signal(
                barrier_sem, inc=1,
                device_id=(nbr,), device_id_type=pl.DeviceIdType.MESH,
            )
        pl.semaphore_wait(barrier_sem, 2)

        wq_comm[0] = wq_ref[...]
        wo_comm[0] = wo_ref[...]

        rb = lax.broadcasted_iota(jnp.int32, (sq_loc, SKV), 0) // 64 + my * HEADS_PER
        cb = lax.broadcasted_iota(jnp.int32, (sq_loc, SKV), 1) // 64
        keep = (rb == cb) | (cb == 0) | (lax.rem(rb + cb, 3) == 0)

        def compute_chunk(slot):
            origin = lax.rem(my - slot + N_DEV, N_DEV)
            wq_c = wq_comm[slot]
            wo_c = wo_comm[slot]
            for b in range(B):
                q = jnp.dot(x_ref[b], wq_c, preferred_element_type=jnp.float32)
                q = (q * 0.125).astype(jnp.bfloat16)
                ctx_cols = []
                for h in range(HEADS_PER):
                    hg = origin * HEADS_PER + h
                    qh = q[:, h * DH:(h + 1) * DH]
                    kh = kt_ref[hg, b]
                    s = nt_dot(qh, kh)
                    s = jnp.where(keep, s, -1e9)
                    s = s - jnp.max(s, axis=1, keepdims=True)
                    w = jnp.exp(s)
                    w = w / jnp.sum(w, axis=1, keepdims=True)
                    vh = vt_ref[hg, b]
                    ctx_cols.append(
                        jnp.dot(w.astype(jnp.bfloat16), vh,
                                preferred_element_type=jnp.float32)
                    )
                ctx = jnp.concatenate(ctx_cols, axis=1).astype(jnp.bfloat16)
                contrib = jnp.dot(ctx, wo_c, preferred_element_type=jnp.float32)
                if slot == 0:
                    out_ref[b] = contrib
                else:
                    out_ref[b] = out_ref[b] + contrib

        for h in range(N_DEV - 1):
            rq = pltpu.make_async_remote_copy(
                src_ref=wq_comm.at[h], dst_ref=wq_comm.at[h + 1],
                send_sem=wq_send.at[h], recv_sem=wq_recv.at[h + 1],
                device_id=(right,), device_id_type=pl.DeviceIdType.MESH,
            )
            ro = pltpu.make_async_remote_copy(
                src_ref=wo_comm.at[h], dst_ref=wo_comm.at[h + 1],
                send_sem=wo_send.at[h], recv_sem=wo_recv.at[h + 1],
                device_id=(right,), device_id_type=pl.DeviceIdType.MESH,
            )
            rq.start()
            ro.start()
            compute_chunk(h)
            rq.wait()
            ro.wait()
        compute_chunk(N_DEV - 1)

    return pl.pallas_call(
        body,
        out_shape=jax.ShapeDtypeStruct((B, sq_loc, d_model), jnp.float32),
        in_specs=[pl.BlockSpec(memory_space=pltpu.VMEM)] * 5,
        out_specs=pl.BlockSpec(memory_space=pltpu.VMEM),
        scratch_shapes=[
            pltpu.VMEM((N_DEV, d_model, d_chunk), jnp.bfloat16),
            pltpu.VMEM((N_DEV, d_chunk, d_model), jnp.bfloat16),
            pltpu.SemaphoreType.DMA((N_DEV,)),
            pltpu.SemaphoreType.DMA((N_DEV,)),
            pltpu.SemaphoreType.DMA((N_DEV,)),
            pltpu.SemaphoreType.DMA((N_DEV,)),
        ],
        compiler_params=pltpu.CompilerParams(collective_id=0),
    )(x_bf, wq_bf, k_t, v_t, wo_bf)


# baseline (device time: 197779 ns/iter reference)
import jax
import jax.numpy as jnp
from jax import lax
from jax.experimental import pallas as pl
from jax.experimental.pallas import tpu as pltpu

N_DEV = 8
HEADS_PER = 8
DH = 64
B = 2
SKV = 512


def kernel(x, Wq, K_ext, V_ext, Wo):
    _, sq_loc, d_model = x.shape
    d_chunk = Wq.shape[1]

    k_t = jnp.transpose(K_ext, (2, 0, 1, 3)).astype(jnp.bfloat16)
    v_t = jnp.transpose(V_ext, (2, 0, 1, 3)).astype(jnp.bfloat16)
    x_bf = x.astype(jnp.bfloat16)
    wq_bf = Wq.astype(jnp.bfloat16)
    wo_bf = Wo.astype(jnp.bfloat16)

    def nt_dot(a, b):
        return lax.dot_general(
            a, b, (((1,), (1,)), ((), ())), preferred_element_type=jnp.float32
        )

    def body(x_ref, wq_ref, kt_ref, vt_ref, wo_ref, out_ref,
             wq_comm, wo_comm, wq_send, wq_recv, wo_send, wo_recv):
        my = lax.axis_index("i")
        left = lax.rem(my + N_DEV - 1, N_DEV)
        right = lax.rem(my + 1, N_DEV)

        barrier_sem = pltpu.get_barrier_semaphore()
        for nbr in (left, right):
            pl.semaphore_signal(
                barrier_sem, inc=1,
                device_id=(nbr,), device_id_type=pl.DeviceIdType.MESH,
            )
        pl.semaphore_wait(barrier_sem, 2)

        wq_comm[0] = wq_ref[...]
        wo_comm[0] = wo_ref[...]

        rb = lax.broadcasted_iota(jnp.int32, (sq_loc, SKV), 0) // 64 + my * HEADS_PER
        cb = lax.broadcasted_iota(jnp.int32, (sq_loc, SKV), 1) // 64
        keep = (rb == cb) | (cb == 0) | (lax.rem(rb + cb, 3) == 0)

        def compute_chunk(slot):
            origin = lax.rem(my - slot + N_DEV, N_DEV)
            wq_c = wq_comm[slot]
            wo_c = wo_comm[slot]
            for b in range(B):
                q = jnp.dot(x_ref[b], wq_c, preferred_element_type=jnp.float32)
                q = (q * 0.125).astype(jnp.bfloat16)
                ctx_cols = []
                for h in range(HEADS_PER):
                    hg = origin * HEADS_PER + h
                    qh = q[:, h * DH:(h + 1) * DH]
                    kh = kt_ref[hg, b]
                    s = nt_dot(qh, kh)
                    s = jnp.where(keep, s, -1e9)
                    s = s - jnp.max(s, axis=1, keepdims=True)
                    w = jnp.exp(s)
                    w = w / jnp.sum(w, axis=1, keepdims=True)
                    vh = vt_ref[hg, b]
                    ctx_cols.append(
                        jnp.dot(w.astype(jnp.bfloat16), vh,
                                preferred_element_type=jnp.float32)
                    )
                ctx = jnp.concatenate(ctx_cols, axis=1).astype(jnp.bfloat16)
                contrib = jnp.dot(ctx, wo_c, preferred_element_type=jnp.float32)
                if slot == 0:
                    out_ref[b] = contrib
                else:
                    out_ref[b] = out_ref[b] + contrib

        for h in range(N_DEV - 1):
            rq = pltpu.make_async_remote_copy(
                src_ref=wq_comm.at[h], dst_ref=wq_comm.at[h + 1],
                send_sem=wq_send.at[h], recv_sem=wq_recv.at[h + 1],
                device_id=(right,), device_id_type=pl.DeviceIdType.MESH,
            )
            ro = pltpu.make_async_remote_copy(
                src_ref=wo_comm.at[h], dst_ref=wo_comm.at[h + 1],
                send_sem=wo_send.at[h], recv_sem=wo_recv.at[h + 1],
                device_id=(right,), device_id_type=pl.DeviceIdType.MESH,
            )
            rq.start()
            ro.start()
            compute_chunk(h)
            rq.wait()
            ro.wait()
        compute_chunk(N_DEV - 1)

    return pl.pallas_call(
        body,
        out_shape=jax.ShapeDtypeStruct((B, sq_loc, d_model), jnp.float32),
        in_specs=[pl.BlockSpec(memory_space=pltpu.VMEM)] * 5,
        out_specs=pl.BlockSpec(memory_space=pltpu.VMEM),
        scratch_shapes=[
            pltpu.VMEM((N_DEV, d_model, d_chunk), jnp.bfloat16),
            pltpu.VMEM((N_DEV, d_chunk, d_model), jnp.bfloat16),
            pltpu.SemaphoreType.DMA((N_DEV,)),
            pltpu.SemaphoreType.DMA((N_DEV,)),
            pltpu.SemaphoreType.DMA((N_DEV,)),
            pltpu.SemaphoreType.DMA((N_DEV,)),
        ],
        compiler_params=pltpu.CompilerParams(
            collective_id=0, vmem_limit_bytes=62 * 1024 * 1024
        ),
    )(x_bf, wq_bf, k_t, v_t, wo_bf)


# device time: 142357 ns/iter; 1.3893x vs baseline; 1.3893x over previous
import jax
import jax.numpy as jnp
from jax import lax
from jax.experimental import pallas as pl
from jax.experimental.pallas import tpu as pltpu

N_DEV = 8
HEADS_PER = 8
DH = 64
B = 2
SKV = 512
R_HOPS = 4
L_HOPS = 3


def kernel(x, Wq, K_ext, V_ext, Wo):
    _, sq_loc, d_model = x.shape
    d_chunk = Wq.shape[1]

    k_t = jnp.transpose(K_ext, (2, 0, 1, 3)).astype(jnp.bfloat16)
    v_t = jnp.transpose(V_ext, (2, 0, 1, 3)).astype(jnp.bfloat16)
    x_bf = x.astype(jnp.bfloat16)
    wq_bf = Wq.astype(jnp.bfloat16)
    wo_bf = Wo.astype(jnp.bfloat16)

    def nt_dot(a, b):
        return lax.dot_general(
            a, b, (((1,), (1,)), ((), ())), preferred_element_type=jnp.float32
        )

    def body(x_ref, wq_ref, kt_ref, vt_ref, wo_ref, out_ref,
             wq_r, wo_r, wq_l, wo_l,
             r_send_q, r_recv_q, r_send_o, r_recv_o,
             l_send_q, l_recv_q, l_send_o, l_recv_o):
        my = lax.axis_index("i")
        left = lax.rem(my + N_DEV - 1, N_DEV)
        right = lax.rem(my + 1, N_DEV)

        barrier_sem = pltpu.get_barrier_semaphore()
        for nbr in (left, right):
            pl.semaphore_signal(
                barrier_sem, inc=1,
                device_id=(nbr,), device_id_type=pl.DeviceIdType.MESH,
            )
        pl.semaphore_wait(barrier_sem, 2)

        wq_r[0] = wq_ref[...]
        wo_r[0] = wo_ref[...]
        wq_l[0] = wq_ref[...]
        wo_l[0] = wo_ref[...]

        rb = lax.broadcasted_iota(jnp.int32, (sq_loc, SKV), 0) // 64 + my * HEADS_PER
        cb = lax.broadcasted_iota(jnp.int32, (sq_loc, SKV), 1) // 64
        keep = (rb == cb) | (cb == 0) | (lax.rem(rb + cb, 3) == 0)
        keepf = keep.astype(jnp.float32)

        def compute_chunk(wq_c, wo_c, origin, first):
            for b in range(B):
                q = jnp.dot(x_ref[b], wq_c, preferred_element_type=jnp.float32)
                q = (q * 0.125).astype(jnp.bfloat16)
                ctx_cols = []
                for h in range(HEADS_PER):
                    hg = origin * HEADS_PER + h
                    qh = q[:, h * DH:(h + 1) * DH]
                    kh = kt_ref[hg, b]
                    s = nt_dot(qh, kh)
                    w = jnp.exp(s) * keepf
                    denom = jnp.sum(w, axis=1, keepdims=True)
                    vh = vt_ref[hg, b]
                    ctx_h = jnp.dot(w.astype(jnp.bfloat16), vh,
                                    preferred_element_type=jnp.float32)
                    ctx_cols.append(ctx_h / denom)
                ctx = jnp.concatenate(ctx_cols, axis=1).astype(jnp.bfloat16)
                contrib = jnp.dot(ctx, wo_c, preferred_element_type=jnp.float32)
                if first:
                    out_ref[b] = contrib
                else:
                    out_ref[b] = out_ref[b] + contrib

        def rdma(buf, send, recv, h, nbr):
            return pltpu.make_async_remote_copy(
                src_ref=buf.at[h], dst_ref=buf.at[h + 1],
                send_sem=send.at[h], recv_sem=recv.at[h + 1],
                device_id=(nbr,), device_id_type=pl.DeviceIdType.MESH,
            )

        for h in range(R_HOPS):
            rq = rdma(wq_r, r_send_q, r_recv_q, h, right)
            ro = rdma(wo_r, r_send_o, r_recv_o, h, right)
            rq.start()
            ro.start()
            if h < L_HOPS:
                lq = rdma(wq_l, l_send_q, l_recv_q, h, left)
                lo = rdma(wo_l, l_send_o, l_recv_o, h, left)
                lq.start()
                lo.start()
            if h == 0:
                compute_chunk(wq_r[0], wo_r[0], my, first=True)
            else:
                compute_chunk(wq_r[h], wo_r[h],
                              lax.rem(my - h + N_DEV, N_DEV), first=False)
                compute_chunk(wq_l[h], wo_l[h],
                              lax.rem(my + h, N_DEV), first=False)
            rq.wait()
            ro.wait()
            if h < L_HOPS:
                lq.wait()
                lo.wait()
        compute_chunk(wq_r[R_HOPS], wo_r[R_HOPS],
                      lax.rem(my - R_HOPS + N_DEV, N_DEV), first=False)

    return pl.pallas_call(
        body,
        out_shape=jax.ShapeDtypeStruct((B, sq_loc, d_model), jnp.float32),
        in_specs=[pl.BlockSpec(memory_space=pltpu.VMEM)] * 5,
        out_specs=pl.BlockSpec(memory_space=pltpu.VMEM),
        scratch_shapes=[
            pltpu.VMEM((R_HOPS + 1, d_model, d_chunk), jnp.bfloat16),
            pltpu.VMEM((R_HOPS + 1, d_chunk, d_model), jnp.bfloat16),
            pltpu.VMEM((L_HOPS + 1, d_model, d_chunk), jnp.bfloat16),
            pltpu.VMEM((L_HOPS + 1, d_chunk, d_model), jnp.bfloat16),
            pltpu.SemaphoreType.DMA((R_HOPS,)),
            pltpu.SemaphoreType.DMA((R_HOPS + 1,)),
            pltpu.SemaphoreType.DMA((R_HOPS,)),
            pltpu.SemaphoreType.DMA((R_HOPS + 1,)),
            pltpu.SemaphoreType.DMA((L_HOPS,)),
            pltpu.SemaphoreType.DMA((L_HOPS + 1,)),
            pltpu.SemaphoreType.DMA((L_HOPS,)),
            pltpu.SemaphoreType.DMA((L_HOPS + 1,)),
        ],
        compiler_params=pltpu.CompilerParams(
            collective_id=0, vmem_limit_bytes=62 * 1024 * 1024
        ),
    )(x_bf, wq_bf, k_t, v_t, wo_bf)


# device time: 128644 ns/iter; 1.5374x vs baseline; 1.1066x over previous
import jax
import jax.numpy as jnp
from jax import lax
from jax.experimental import pallas as pl
from jax.experimental.pallas import tpu as pltpu

N_DEV = 8
HEADS_PER = 8
DH = 64
B = 2
SKV = 512
R_HOPS = 4
L_HOPS = 3


def kernel(x, Wq, K_ext, V_ext, Wo):
    _, sq_loc, d_model = x.shape
    d_chunk = Wq.shape[1]

    k_t = K_ext.reshape(B, SKV, HEADS_PER * N_DEV * DH).astype(jnp.bfloat16)
    v_t = V_ext.reshape(B, SKV, HEADS_PER * N_DEV * DH).astype(jnp.bfloat16)
    x_bf = x.astype(jnp.bfloat16)
    wq_bf = Wq.astype(jnp.bfloat16)
    wo_bf = Wo.astype(jnp.bfloat16)

    def nt_dot(a, b, out_dtype):
        return lax.dot_general(
            a, b, (((1,), (1,)), ((), ())), preferred_element_type=out_dtype
        )

    def body(x_ref, wq_ref, kt_ref, vt_ref, wo_ref, out_ref,
             wq_r, wo_r, wq_l, wo_l,
             r_send_q, r_recv_q, r_send_o, r_recv_o,
             l_send_q, l_recv_q, l_send_o, l_recv_o):
        my = lax.axis_index("i")
        left = lax.rem(my + N_DEV - 1, N_DEV)
        right = lax.rem(my + 1, N_DEV)

        barrier_sem = pltpu.get_barrier_semaphore()
        for nbr in (left, right):
            pl.semaphore_signal(
                barrier_sem, inc=1,
                device_id=(nbr,), device_id_type=pl.DeviceIdType.MESH,
            )
        pl.semaphore_wait(barrier_sem, 2)

        wq_r[0] = wq_ref[...]
        wo_r[0] = wo_ref[...]
        wq_l[0] = wq_ref[...]
        wo_l[0] = wo_ref[...]

        rb = lax.broadcasted_iota(jnp.int32, (sq_loc, SKV), 0) // 64 + my * HEADS_PER
        cb = lax.broadcasted_iota(jnp.int32, (sq_loc, SKV), 1) // 64
        keep = (rb == cb) | (cb == 0) | (lax.rem(rb + cb, 3) == 0)
        keepb = keep.astype(jnp.bfloat16)

        def compute_chunk(wq_c, wo_c, origin, first):
            for b in range(B):
                q = jnp.dot(x_ref[b], wq_c, preferred_element_type=jnp.float32)
                q = (q * 0.125).astype(jnp.bfloat16)
                kc = kt_ref[b, :, pl.ds(origin * d_chunk, d_chunk)]
                vc = vt_ref[b, :, pl.ds(origin * d_chunk, d_chunk)]
                ctx_cols = []
                for h in range(HEADS_PER):
                    qh = q[:, h * DH:(h + 1) * DH]
                    kh = kc[:, h * DH:(h + 1) * DH]
                    s = nt_dot(qh, kh, jnp.float32)
                    w = jnp.exp(s.astype(jnp.bfloat16)) * keepb
                    denom = jnp.sum(w.astype(jnp.float32), axis=1, keepdims=True)
                    vh = vc[:, h * DH:(h + 1) * DH]
                    ctx_h = jnp.dot(w, vh, preferred_element_type=jnp.float32)
                    ctx_cols.append(ctx_h / denom)
                ctx = jnp.concatenate(ctx_cols, axis=1).astype(jnp.bfloat16)
                contrib = jnp.dot(ctx, wo_c, preferred_element_type=jnp.float32)
                if first:
                    out_ref[b] = contrib
                else:
                    out_ref[b] = out_ref[b] + contrib

        def rdma(buf, send, recv, h, nbr):
            return pltpu.make_async_remote_copy(
                src_ref=buf.at[h], dst_ref=buf.at[h + 1],
                send_sem=send.at[h], recv_sem=recv.at[h + 1],
                device_id=(nbr,), device_id_type=pl.DeviceIdType.MESH,
            )

        for h in range(R_HOPS):
            rq = rdma(wq_r, r_send_q, r_recv_q, h, right)
            ro = rdma(wo_r, r_send_o, r_recv_o, h, right)
            rq.start()
            ro.start()
            if h < L_HOPS:
                lq = rdma(wq_l, l_send_q, l_recv_q, h, left)
                lo = rdma(wo_l, l_send_o, l_recv_o, h, left)
                lq.start()
                lo.start()
            if h == 0:
                compute_chunk(wq_r[0], wo_r[0], my, first=True)
            else:
                compute_chunk(wq_r[h], wo_r[h],
                              lax.rem(my - h + N_DEV, N_DEV), first=False)
                compute_chunk(wq_l[h], wo_l[h],
                              lax.rem(my + h, N_DEV), first=False)
            rq.wait()
            ro.wait()
            if h < L_HOPS:
                lq.wait()
                lo.wait()
        compute_chunk(wq_r[R_HOPS], wo_r[R_HOPS],
                      lax.rem(my - R_HOPS + N_DEV, N_DEV), first=False)

    return pl.pallas_call(
        body,
        out_shape=jax.ShapeDtypeStruct((B, sq_loc, d_model), jnp.float32),
        in_specs=[pl.BlockSpec(memory_space=pltpu.VMEM)] * 5,
        out_specs=pl.BlockSpec(memory_space=pltpu.VMEM),
        scratch_shapes=[
            pltpu.VMEM((R_HOPS + 1, d_model, d_chunk), jnp.bfloat16),
            pltpu.VMEM((R_HOPS + 1, d_chunk, d_model), jnp.bfloat16),
            pltpu.VMEM((L_HOPS + 1, d_model, d_chunk), jnp.bfloat16),
            pltpu.VMEM((L_HOPS + 1, d_chunk, d_model), jnp.bfloat16),
            pltpu.SemaphoreType.DMA((R_HOPS,)),
            pltpu.SemaphoreType.DMA((R_HOPS + 1,)),
            pltpu.SemaphoreType.DMA((R_HOPS,)),
            pltpu.SemaphoreType.DMA((R_HOPS + 1,)),
            pltpu.SemaphoreType.DMA((L_HOPS,)),
            pltpu.SemaphoreType.DMA((L_HOPS + 1,)),
            pltpu.SemaphoreType.DMA((L_HOPS,)),
            pltpu.SemaphoreType.DMA((L_HOPS + 1,)),
        ],
        compiler_params=pltpu.CompilerParams(
            collective_id=0, vmem_limit_bytes=62 * 1024 * 1024
        ),
    )(x_bf, wq_bf, k_t, v_t, wo_bf)


# device time: 127250 ns/iter; 1.5543x vs baseline; 1.0110x over previous
import jax
import jax.numpy as jnp
from jax import lax
from jax.experimental import pallas as pl
from jax.experimental.pallas import tpu as pltpu

N_DEV = 8
HEADS_PER = 8
DH = 64
B = 2
SKV = 512
R_HOPS = 4
L_HOPS = 3


def kernel(x, Wq, K_ext, V_ext, Wo):
    _, sq_loc, d_model = x.shape
    d_chunk = Wq.shape[1]

    k_t = K_ext.reshape(B, SKV, HEADS_PER * N_DEV * DH).astype(jnp.bfloat16)
    v_t = V_ext.reshape(B, SKV, HEADS_PER * N_DEV * DH).astype(jnp.bfloat16)

    def nt_dot(a, b):
        return lax.dot_general(
            a, b, (((1,), (1,)), ((), ())), preferred_element_type=jnp.float32
        )

    def body(x_ref, wq_ref, kt_ref, vt_ref, wo_ref, out_ref,
             x_bf, wq_r, wo_r, wq_l, wo_l,
             r_send_q, r_recv_q, r_send_o, r_recv_o,
             l_send_q, l_recv_q, l_send_o, l_recv_o):
        my = lax.axis_index("i")
        left = lax.rem(my + N_DEV - 1, N_DEV)
        right = lax.rem(my + 1, N_DEV)

        x_bf[...] = x_ref[...].astype(jnp.bfloat16)
        wq_r[0] = wq_ref[...].astype(jnp.bfloat16)
        wo_r[0] = wo_ref[...].astype(jnp.bfloat16)
        wq_l[0] = wq_r[0]
        wo_l[0] = wo_r[0]

        barrier_sem = pltpu.get_barrier_semaphore()
        for nbr in (left, right):
            pl.semaphore_signal(
                barrier_sem, inc=1,
                device_id=(nbr,), device_id_type=pl.DeviceIdType.MESH,
            )
        pl.semaphore_wait(barrier_sem, 2)

        rb = lax.broadcasted_iota(jnp.int32, (sq_loc, SKV), 0) // 64 + my * HEADS_PER
        cb = lax.broadcasted_iota(jnp.int32, (sq_loc, SKV), 1) // 64
        keep = (rb == cb) | (cb == 0) | (lax.rem(rb + cb, 3) == 0)
        keepb = keep.astype(jnp.bfloat16)

        def compute_chunk(wq_c, wo_c, origin, first):
            for b in range(B):
                q = jnp.dot(x_bf[b], wq_c, preferred_element_type=jnp.float32)
                q = (q * 0.125).astype(jnp.bfloat16)
                kc = kt_ref[b, :, pl.ds(origin * d_chunk, d_chunk)]
                vc = vt_ref[b, :, pl.ds(origin * d_chunk, d_chunk)]
                ctx_cols = []
                for h in range(HEADS_PER):
                    qh = q[:, h * DH:(h + 1) * DH]
                    kh = kc[:, h * DH:(h + 1) * DH]
                    s = nt_dot(qh, kh)
                    w = jnp.exp(s.astype(jnp.bfloat16)) * keepb
                    denom = jnp.sum(w.astype(jnp.float32), axis=1, keepdims=True)
                    vh = vc[:, h * DH:(h + 1) * DH]
                    ctx_h = jnp.dot(w, vh, preferred_element_type=jnp.float32)
                    ctx_cols.append(ctx_h / denom)
                ctx = jnp.concatenate(ctx_cols, axis=1).astype(jnp.bfloat16)
                contrib = jnp.dot(ctx, wo_c, preferred_element_type=jnp.float32)
                if first:
                    out_ref[b] = contrib
                else:
                    out_ref[b] = out_ref[b] + contrib

        def rdma(buf, send, recv, h, nbr):
            return pltpu.make_async_remote_copy(
                src_ref=buf.at[h], dst_ref=buf.at[h + 1],
                send_sem=send.at[h], recv_sem=recv.at[h + 1],
                device_id=(nbr,), device_id_type=pl.DeviceIdType.MESH,
            )

        for h in range(R_HOPS):
            rq = rdma(wq_r, r_send_q, r_recv_q, h, right)
            ro = rdma(wo_r, r_send_o, r_recv_o, h, right)
            rq.start()
            ro.start()
            if h < L_HOPS:
                lq = rdma(wq_l, l_send_q, l_recv_q, h, left)
                lo = rdma(wo_l, l_send_o, l_recv_o, h, left)
                lq.start()
                lo.start()
            if h == 0:
                compute_chunk(wq_r[0], wo_r[0], my, first=True)
            else:
                compute_chunk(wq_r[h], wo_r[h],
                              lax.rem(my - h + N_DEV, N_DEV), first=False)
                compute_chunk(wq_l[h], wo_l[h],
                              lax.rem(my + h, N_DEV), first=False)
            rq.wait()
            ro.wait()
            if h < L_HOPS:
                lq.wait()
                lo.wait()
        compute_chunk(wq_r[R_HOPS], wo_r[R_HOPS],
                      lax.rem(my - R_HOPS + N_DEV, N_DEV), first=False)

    return pl.pallas_call(
        body,
        out_shape=jax.ShapeDtypeStruct((B, sq_loc, d_model), jnp.float32),
        in_specs=[pl.BlockSpec(memory_space=pltpu.VMEM)] * 5,
        out_specs=pl.BlockSpec(memory_space=pltpu.VMEM),
        scratch_shapes=[
            pltpu.VMEM((B, sq_loc, d_model), jnp.bfloat16),
            pltpu.VMEM((R_HOPS + 1, d_model, d_chunk), jnp.bfloat16),
            pltpu.VMEM((R_HOPS + 1, d_chunk, d_model), jnp.bfloat16),
            pltpu.VMEM((L_HOPS + 1, d_model, d_chunk), jnp.bfloat16),
            pltpu.VMEM((L_HOPS + 1, d_chunk, d_model), jnp.bfloat16),
            pltpu.SemaphoreType.DMA((R_HOPS,)),
            pltpu.SemaphoreType.DMA((R_HOPS + 1,)),
            pltpu.SemaphoreType.DMA((R_HOPS,)),
            pltpu.SemaphoreType.DMA((R_HOPS + 1,)),
            pltpu.SemaphoreType.DMA((L_HOPS,)),
            pltpu.SemaphoreType.DMA((L_HOPS + 1,)),
            pltpu.SemaphoreType.DMA((L_HOPS,)),
            pltpu.SemaphoreType.DMA((L_HOPS + 1,)),
        ],
        compiler_params=pltpu.CompilerParams(
            collective_id=0, vmem_limit_bytes=62 * 1024 * 1024
        ),
    )(x, Wq, k_t, v_t, Wo)
